# baseline (speedup 1.0000x reference)
import functools

import jax
import jax.numpy as jnp
from jax import lax
from jax.experimental import pallas as pl
from jax.experimental.pallas import tpu as pltpu
from jax.experimental.pallas import tpu_sc as plsc

N = 10000
E = 320000
D = 128

NC = 2
NS = 16
NW = NC * NS

CHUNK = 128
CH0 = 160
CH1 = 0
SLAB = 16
E_PAD = NS * (CH0 + CH1) * CHUNK
DUMP = N
N_PAD = 10112
ROWS_PER_TILE = N_PAD // NS


def _sc_body(x_hbm, col0_hbm, row0_hbm, acc_hbm,
             col_v, row_v, rows_a, rows_b, acc_s, sem_a, sem_b):
    cid = lax.axis_index("c")
    sid = lax.axis_index("s")

    zeros16 = jnp.zeros((16,), jnp.float32)

    def zloop(i, carry):
        r = i // (D // 16)
        c = (i % (D // 16)) * 16
        rows_a[r, pl.ds(c, 16)] = zeros16
        return carry

    base = sid * ROWS_PER_TILE

    @pl.when(cid == 0)
    def _():
        lax.fori_loop(0, CHUNK * (D // 16), zloop, 0)
        for k in range(ROWS_PER_TILE // CHUNK):
            pltpu.sync_copy(rows_a, acc_s.at[pl.ds(base + k * CHUNK, CHUNK)])
        rem = ROWS_PER_TILE % CHUNK
        if rem:
            off = ROWS_PER_TILE - rem
            pltpu.sync_copy(rows_a.at[pl.ds(0, rem)],
                            acc_s.at[pl.ds(base + off, rem)])

    plsc.subcore_barrier()

    def run_edges(col_hbm, row_hbm, n_slabs):
        for s in range(n_slabs):
            pltpu.sync_copy(col_hbm.at[sid, pl.ds(s * SLAB, SLAB)], col_v)
            pltpu.sync_copy(row_hbm.at[sid, pl.ds(s * SLAB, SLAB)], row_v)
            pltpu.async_copy(x_hbm.at[col_v.at[0]], rows_a, sem_a)
            pltpu.async_copy(x_hbm.at[col_v.at[1]], rows_b, sem_b)

            def mloop(j2, carry):
                j = j2 * 2
                pltpu.make_async_copy(x_hbm.at[col_v.at[j]], rows_a, sem_a).wait()
                pltpu.sync_copy(rows_a, acc_s.at[row_v.at[j]], add=True)
                pltpu.async_copy(x_hbm.at[col_v.at[j + 2]], rows_a, sem_a)
                pltpu.make_async_copy(x_hbm.at[col_v.at[j + 1]], rows_b, sem_b).wait()
                pltpu.sync_copy(rows_b, acc_s.at[row_v.at[j + 1]], add=True)
                pltpu.async_copy(x_hbm.at[col_v.at[j + 3]], rows_b, sem_b)
                return carry

            lax.fori_loop(0, SLAB // 2 - 1, mloop, 0)
            j = SLAB - 2
            pltpu.make_async_copy(x_hbm.at[col_v.at[j]], rows_a, sem_a).wait()
            pltpu.sync_copy(rows_a, acc_s.at[row_v.at[j]], add=True)
            pltpu.make_async_copy(x_hbm.at[col_v.at[j + 1]], rows_b, sem_b).wait()
            pltpu.sync_copy(rows_b, acc_s.at[row_v.at[j + 1]], add=True)

    @pl.when(cid == 0)
    def _():
        run_edges(col0_hbm, row0_hbm, CH0 // SLAB)

    plsc.subcore_barrier()

    @pl.when(cid == 0)
    def _():
        pltpu.sync_copy(acc_s.at[pl.ds(base, ROWS_PER_TILE)],
                        acc_hbm.at[0, pl.ds(base, ROWS_PER_TILE)])


_sc_spmm = functools.partial(
    pl.kernel,
    out_type=jax.ShapeDtypeStruct((1, N_PAD, D), jnp.float32),
    mesh=plsc.VectorSubcoreMesh(core_axis_name="c", subcore_axis_name="s"),
    scratch_types=[
        pltpu.VMEM((SLAB, CHUNK), jnp.int32),
        pltpu.VMEM((SLAB, CHUNK), jnp.int32),
        pltpu.VMEM((CHUNK, D), jnp.float32),
        pltpu.VMEM((CHUNK, D), jnp.float32),
        pltpu.VMEM_SHARED((N_PAD, D), jnp.float32),
        pltpu.SemaphoreType.DMA,
        pltpu.SemaphoreType.DMA,
    ],
)(_sc_body)


def _tc_finalize_body(x_ref, acc_ref, wz_ref, wf_ref, o_ref):
    wz = jnp.maximum(wz_ref[0, 0], 0.0)
    wf = jnp.maximum(wf_ref[0, 0], 0.0)
    tot = wz + wf + 1e-6
    ctx = acc_ref[0]
    o_ref[...] = (wz / tot) * x_ref[...] + (wf / tot) * ctx


def _tc_finalize(x, acc, wz, wf):
    br = 1000
    grid = (N // br,)
    return pl.pallas_call(
        _tc_finalize_body,
        out_shape=jax.ShapeDtypeStruct((N, D), jnp.float32),
        grid=grid,
        in_specs=[
            pl.BlockSpec((br, D), lambda i: (i, 0)),
            pl.BlockSpec((1, br, D), lambda i: (0, i, 0)),
            pl.BlockSpec((1, 1), lambda i: (0, 0)),
            pl.BlockSpec((1, 1), lambda i: (0, 0)),
        ],
        out_specs=pl.BlockSpec((br, D), lambda i: (i, 0)),
    )(x, acc, wz, wf)


def kernel(index_vectors, edge_index, w_zeroth, w_fst):
    row = edge_index[0]
    col = edge_index[1]
    pad = E_PAD - E
    col_p = jnp.concatenate([col, jnp.zeros((pad,), jnp.int32)])
    row_p = jnp.concatenate([row, jnp.full((pad,), DUMP, jnp.int32)])
    col0 = col_p.reshape(NS, CH0, CHUNK)
    row0 = row_p.reshape(NS, CH0, CHUNK)
    acc = _sc_spmm(index_vectors, col0, row0)
    return _tc_finalize(index_vectors, acc, w_zeroth, w_fst)


# speedup vs baseline: 1.7798x; 1.7798x over previous
import functools

import jax
import jax.numpy as jnp
from jax import lax
from jax.experimental import pallas as pl
from jax.experimental.pallas import tpu as pltpu
from jax.experimental.pallas import tpu_sc as plsc

N = 10000
E = 320000
D = 128

NC = 2
NS = 16

CHUNK = 32
SLAB = 2
NCH = 640
NSLAB = NCH // SLAB
E_PAD = NS * NCH * CHUNK

HALF = 5000
ACC_ROWS = 5120
DUMP_BASE = 5056
ACC_PER_TILE = ACC_ROWS // NS
X_PER_TILE = 632


def _sc_body(x_hbm, col_hbm, row0_hbm, row1_hbm, acc_hbm,
             colA, rowA, colB, rowB, b0, b1, xs, acc_s,
             sg0, sg1, si0, si1):
    cid = lax.axis_index("c")
    sid = lax.axis_index("s")

    xbase = sid * X_PER_TILE

    @pl.when(sid < NS - 1)
    def _():
        pltpu.sync_copy(x_hbm.at[pl.ds(xbase, X_PER_TILE)],
                        xs.at[pl.ds(xbase, X_PER_TILE)])

    @pl.when(sid == NS - 1)
    def _():
        pltpu.sync_copy(x_hbm.at[pl.ds((NS - 1) * X_PER_TILE, 520)],
                        xs.at[pl.ds((NS - 1) * X_PER_TILE, 520)])

    z16 = jnp.zeros((16,), jnp.float32)

    def zloop(i, c):
        r = i // (D // 16)
        cc = (i % (D // 16)) * 16
        b0[r, pl.ds(cc, 16)] = z16
        return c

    lax.fori_loop(0, CHUNK * (D // 16), zloop, 0)
    abase = sid * ACC_PER_TILE
    for k in range(ACC_PER_TILE // CHUNK):
        pltpu.sync_copy(b0, acc_s.at[pl.ds(abase + k * CHUNK, CHUNK)])

    plsc.subcore_barrier()

    def run(row_hbm):
        pltpu.sync_copy(col_hbm.at[sid, pl.ds(0, SLAB)], colA)
        pltpu.sync_copy(row_hbm.at[sid, pl.ds(0, SLAB)], rowA)
        pltpu.async_copy(col_hbm.at[sid, pl.ds(SLAB, SLAB)], colB, si1)
        pltpu.async_copy(row_hbm.at[sid, pl.ds(SLAB, SLAB)], rowB, si1)
        pltpu.async_copy(xs.at[colA.at[0]], b0, sg0)
        pltpu.async_copy(xs.at[colA.at[1]], b1, sg1)

        def pair(s2, carry):
            nA = (2 * s2 + 2) * SLAB
            nB = (2 * s2 + 3) * SLAB
            pltpu.make_async_copy(xs.at[colA.at[0]], b0, sg0).wait()
            pltpu.sync_copy(b0, acc_s.at[rowA.at[0]], add=True)
            pltpu.make_async_copy(col_hbm.at[sid, pl.ds(nB - 2 * SLAB, SLAB)],
                                  colB, si1).wait()
            pltpu.make_async_copy(row_hbm.at[sid, pl.ds(nB - 2 * SLAB, SLAB)],
                                  rowB, si1).wait()
            pltpu.async_copy(xs.at[colB.at[0]], b0, sg0)
            pltpu.make_async_copy(xs.at[colA.at[1]], b1, sg1).wait()
            pltpu.sync_copy(b1, acc_s.at[rowA.at[1]], add=True)
            pltpu.async_copy(xs.at[colB.at[1]], b1, sg1)
            pltpu.async_copy(col_hbm.at[sid, pl.ds(nA, SLAB)], colA, si0)
            pltpu.async_copy(row_hbm.at[sid, pl.ds(nA, SLAB)], rowA, si0)
            pltpu.make_async_copy(xs.at[colB.at[0]], b0, sg0).wait()
            pltpu.sync_copy(b0, acc_s.at[rowB.at[0]], add=True)
            pltpu.make_async_copy(col_hbm.at[sid, pl.ds(nA, SLAB)],
                                  colA, si0).wait()
            pltpu.make_async_copy(row_hbm.at[sid, pl.ds(nA, SLAB)],
                                  rowA, si0).wait()
            pltpu.async_copy(xs.at[colA.at[0]], b0, sg0)
            pltpu.make_async_copy(xs.at[colB.at[1]], b1, sg1).wait()
            pltpu.sync_copy(b1, acc_s.at[rowB.at[1]], add=True)
            pltpu.async_copy(xs.at[colA.at[1]], b1, sg1)
            pltpu.async_copy(col_hbm.at[sid, pl.ds(nB, SLAB)], colB, si1)
            pltpu.async_copy(row_hbm.at[sid, pl.ds(nB, SLAB)], rowB, si1)
            return carry

        lax.fori_loop(0, NSLAB // 2 - 1, pair, 0)

        pltpu.make_async_copy(xs.at[colA.at[0]], b0, sg0).wait()
        pltpu.sync_copy(b0, acc_s.at[rowA.at[0]], add=True)
        pltpu.make_async_copy(
            col_hbm.at[sid, pl.ds((NSLAB - 1) * SLAB, SLAB)], colB, si1).wait()
        pltpu.make_async_copy(
            row_hbm.at[sid, pl.ds((NSLAB - 1) * SLAB, SLAB)], rowB, si1).wait()
        pltpu.async_copy(xs.at[colB.at[0]], b0, sg0)
        pltpu.make_async_copy(xs.at[colA.at[1]], b1, sg1).wait()
        pltpu.sync_copy(b1, acc_s.at[rowA.at[1]], add=True)
        pltpu.async_copy(xs.at[colB.at[1]], b1, sg1)
        pltpu.make_async_copy(xs.at[colB.at[0]], b0, sg0).wait()
        pltpu.sync_copy(b0, acc_s.at[rowB.at[0]], add=True)
        pltpu.make_async_copy(xs.at[colB.at[1]], b1, sg1).wait()
        pltpu.sync_copy(b1, acc_s.at[rowB.at[1]], add=True)

    @pl.when(cid == 0)
    def _():
        run(row0_hbm)

    @pl.when(cid == 1)
    def _():
        run(row1_hbm)

    plsc.subcore_barrier()

    pltpu.sync_copy(acc_s.at[pl.ds(abase, ACC_PER_TILE)],
                    acc_hbm.at[cid, pl.ds(abase, ACC_PER_TILE)])


_sc_spmm = functools.partial(
    pl.kernel,
    out_type=jax.ShapeDtypeStruct((NC, ACC_ROWS, D), jnp.float32),
    mesh=plsc.VectorSubcoreMesh(core_axis_name="c", subcore_axis_name="s"),
    scratch_types=[
        pltpu.VMEM((SLAB, CHUNK), jnp.int32),
        pltpu.VMEM((SLAB, CHUNK), jnp.int32),
        pltpu.VMEM((SLAB, CHUNK), jnp.int32),
        pltpu.VMEM((SLAB, CHUNK), jnp.int32),
        pltpu.VMEM((CHUNK, D), jnp.float32),
        pltpu.VMEM((CHUNK, D), jnp.float32),
        pltpu.VMEM_SHARED((N, D), jnp.float32),
        pltpu.VMEM_SHARED((ACC_ROWS, D), jnp.float32),
        pltpu.SemaphoreType.DMA,
        pltpu.SemaphoreType.DMA,
        pltpu.SemaphoreType.DMA,
        pltpu.SemaphoreType.DMA,
    ],
)(_sc_body)


def _tc_finalize_body(x_ref, acc_ref, wz_ref, wf_ref, o_ref):
    wz = jnp.maximum(wz_ref[0, 0], 0.0)
    wf = jnp.maximum(wf_ref[0, 0], 0.0)
    tot = wz + wf + 1e-6
    ctx = acc_ref[0]
    o_ref[...] = (wz / tot) * x_ref[...] + (wf / tot) * ctx


def _tc_finalize(x, acc, wz, wf):
    br = 1000
    grid = (N // br,)
    return pl.pallas_call(
        _tc_finalize_body,
        out_shape=jax.ShapeDtypeStruct((N, D), jnp.float32),
        grid=grid,
        in_specs=[
            pl.BlockSpec((br, D), lambda i: (i, 0)),
            pl.BlockSpec((1, br, D), lambda i: (i // 5, i % 5, 0)),
            pl.BlockSpec((1, 1), lambda i: (0, 0)),
            pl.BlockSpec((1, 1), lambda i: (0, 0)),
        ],
        out_specs=pl.BlockSpec((br, D), lambda i: (i, 0)),
    )(x, acc, wz, wf)


def kernel(index_vectors, edge_index, w_zeroth, w_fst):
    row = edge_index[0]
    col = edge_index[1]
    pad = E_PAD - E
    padc = (jnp.arange(pad, dtype=jnp.int32) * 37) % N
    dump = DUMP_BASE + (jnp.arange(E, dtype=jnp.int32) % (ACC_ROWS - DUMP_BASE))
    dumpp = DUMP_BASE + (jnp.arange(pad, dtype=jnp.int32) % (ACC_ROWS - DUMP_BASE))
    col_p = jnp.concatenate([col, padc]).reshape(NS, NCH, CHUNK)
    row0 = jnp.concatenate(
        [jnp.where(row < HALF, row, dump), dumpp]).reshape(NS, NCH, CHUNK)
    row1 = jnp.concatenate(
        [jnp.where(row >= HALF, row - HALF, dump), dumpp]).reshape(NS, NCH, CHUNK)
    acc = _sc_spmm(index_vectors, col_p, row0, row1)
    return _tc_finalize(index_vectors, acc, w_zeroth, w_fst)
